# baseline (speedup 1.0000x reference)
import functools

import jax
import jax.numpy as jnp
from jax import lax
from jax.experimental import pallas as pl
from jax.experimental.pallas import tpu as pltpu
from jax.experimental.pallas import tpu_sc as plsc

F = 26
V = 100000
D = 64
A = 128
B = 4096
FD = F * D
TEMP = 5.0
REG_WEIGHT = 0.1

_info = plsc.get_sparse_core_info()
_NC, _NS = _info.num_cores, _info.num_subcores
_NW = _NC * _NS
_BW = B // _NW


def _sc_gather(tables_flat, idx_flat):
    mesh = plsc.VectorSubcoreMesh(core_axis_name="c", subcore_axis_name="s")

    @functools.partial(
        pl.kernel,
        mesh=mesh,
        out_type=jax.ShapeDtypeStruct((B, FD), jnp.float32),
        scratch_types=[
            pltpu.VMEM((F, _BW), jnp.int32),
            pltpu.VMEM((_BW, D), jnp.float32),
            pltpu.SemaphoreType.DMA,
        ],
        compiler_params=pltpu.CompilerParams(use_tc_tiling_on_sc=False),
    )
    def k(tables_hbm, idx_hbm, x_hbm, idx_v, rows_v, sem):
        wid = lax.axis_index("s") * _NC + lax.axis_index("c")
        base = wid * _BW
        pltpu.sync_copy(idx_hbm.at[:, pl.ds(base, _BW)], idx_v)

        def body(f, carry):
            pltpu.async_copy(tables_hbm.at[idx_v.at[f]], rows_v, sem).wait()
            pltpu.sync_copy(rows_v, x_hbm.at[pl.ds(base, _BW), pl.ds(f * D, D)])
            return carry

        lax.fori_loop(0, F, body, 0)

    return k(tables_flat, idx_flat)


_BT = 512


def _tc_body(x_ref, w_ref, th_ref, out_ref, fs_ref):
    g = jax.nn.sigmoid(th_ref[...] * TEMP)
    xg = x_ref[...] * g
    out_ref[...] = jnp.dot(xg, w_ref[...], preferred_element_type=jnp.float32)

    @pl.when(pl.program_id(0) == 0)
    def _():
        m = jnp.mean(g)
        fs = jnp.mean(g - jnp.abs(g - m)) * REG_WEIGHT
        fs_ref[...] = jnp.full((1, 1), fs, jnp.float32)


def _tc_matmul(x, weight, theta):
    return pl.pallas_call(
        _tc_body,
        grid=(B // _BT,),
        in_specs=[
            pl.BlockSpec((_BT, FD), lambda i: (i, 0)),
            pl.BlockSpec((FD, A), lambda i: (0, 0)),
            pl.BlockSpec((1, FD), lambda i: (0, 0)),
        ],
        out_specs=[
            pl.BlockSpec((_BT, A), lambda i: (i, 0)),
            pl.BlockSpec((1, 1), lambda i: (0, 0)),
        ],
        out_shape=[
            jax.ShapeDtypeStruct((B, A), jnp.float32),
            jax.ShapeDtypeStruct((1, 1), jnp.float32),
        ],
    )(x, weight, theta.reshape(1, FD))


def kernel(inputs, tables, theta, weight):
    idx_flat = inputs.T + (jnp.arange(F, dtype=jnp.int32) * V)[:, None]
    tables_flat = tables.reshape(F * V, D)
    x = _sc_gather(tables_flat, idx_flat)
    out, fs = _tc_matmul(x, weight, theta)
    return (out, fs.reshape(()))
